# baseline (speedup 1.0000x reference)
import jax
import jax.numpy as jnp
import numpy as np
from jax import lax
from jax.experimental import pallas as pl
from jax.experimental.pallas import tpu as pltpu
from jax.experimental.pallas import tpu_sc as plsc

TABLE_SIZE = 524288
NUM_LEVELS = 16
MIN_RESOLUTION = 16
MAX_RESOLUTION = 2048
FEATURE_DIM = 2
N_POINTS = 131072

_K1 = int(np.uint32(2654435761).view(np.int32))
_K2 = 805459861
_MASK = TABLE_SIZE - 1

NW = 32
NPW = N_POINTS // NW
P = 512
NCHUNK = NPW // P
NG = P // 16
OUTD = NUM_LEVELS * FEATURE_DIM
NIDX = 8 * P


def _body(xt_hbm, tpk_hbm, scal_hbm, out_hbm,
          shared, xv, sv, wv0, wv1, idx0, idx1, rows0, rows1, outv,
          sem0, sem1, sem_stage):
    cid = lax.axis_index("c")
    sid = lax.axis_index("s")
    wid = sid * 2 + cid
    base_w = wid * NPW

    pltpu.sync_copy(scal_hbm, sv)
    for c in range(3):
        pltpu.sync_copy(xt_hbm.at[pl.ds(c * N_POINTS + base_w, NPW)],
                        xv.at[pl.ds(c * NPW, NPW)])

    @pl.when(sid == 0)
    def _():
        pltpu.async_copy(tpk_hbm.at[pl.ds(0, TABLE_SIZE)],
                         shared.at[pl.ds(0, TABLE_SIZE)], sem_stage)

    def hashp(l, parity, ci, idxb, wvb):
        off = ci * P
        sbase = parity * TABLE_SIZE

        @plsc.parallel_loop(0, NG, 1, unroll=8)
        def grp1(g):
            p0 = off + g * 16
            x0 = xv[pl.ds(p0, 16)]
            x1 = xv[pl.ds(NPW + p0, 16)]
            x2 = xv[pl.ds(2 * NPW + p0, 16)]
            s = sv[pl.ds(l * 16, 16)]
            sx0 = x0 * s
            sx1 = x1 * s
            sx2 = x2 * s
            f0 = sx0.astype(jnp.int32)
            f1 = sx1.astype(jnp.int32)
            f2 = sx2.astype(jnp.int32)
            ff0 = f0.astype(jnp.float32)
            ff1 = f1.astype(jnp.float32)
            ff2 = f2.astype(jnp.float32)
            c0 = jnp.where(sx0 > ff0, f0 + 1, f0)
            c1 = jnp.where(sx1 > ff1, f1 + 1, f1)
            c2 = jnp.where(sx2 > ff2, f2 + 1, f2)
            wvb[pl.ds(0 * P + g * 16, 16)] = sx0 - ff0
            wvb[pl.ds(1 * P + g * 16, 16)] = sx1 - ff1
            wvb[pl.ds(2 * P + g * 16, 16)] = sx2 - ff2
            tyc = c1 * _K1
            tyf = f1 * _K1
            tzc = c2 * _K2
            tzf = f2 * _K2
            xy_cc = c0 ^ tyc
            xy_cf = c0 ^ tyf
            xy_fc = f0 ^ tyc
            xy_ff = f0 ^ tyf
            hs = [
                xy_cc ^ tzc, xy_cc ^ tzf, xy_cf ^ tzc, xy_fc ^ tzc,
                xy_cf ^ tzf, xy_fc ^ tzf, xy_ff ^ tzc, xy_ff ^ tzf,
            ]
            for k in range(8):
                idxb[pl.ds(k * P + g * 16, 16)] = (hs[k] & _MASK) + sbase

    def interp(l, ci, rowsb, wvb):
        @plsc.parallel_loop(0, NG, 1, unroll=8)
        def grp2(g):
            wx = wvb[pl.ds(0 * P + g * 16, 16)]
            wy = wvb[pl.ds(1 * P + g * 16, 16)]
            wz = wvb[pl.ds(2 * P + g * 16, 16)]
            fpk = [rowsb[pl.ds(k * P + g * 16, 16)] for k in range(8)]
            for ch in range(2):
                if ch == 0:
                    f = [plsc.bitcast(v << 16, jnp.float32) for v in fpk]
                else:
                    f = [plsc.bitcast(v & (-65536), jnp.float32) for v in fpk]
                f03 = f[3] + wx * (f[0] - f[3])
                f12 = f[2] + wx * (f[1] - f[2])
                f56 = f[6] + wx * (f[5] - f[6])
                f47 = f[7] + wx * (f[4] - f[7])
                f0312 = f12 + wy * (f03 - f12)
                f4756 = f56 + wy * (f47 - f56)
                enc = f4756 + wz * (f0312 - f4756)
                outv[ch, pl.ds(g * 16, 16)] = enc
        pltpu.sync_copy(
            outv, out_hbm.at[pl.ds(2 * l, 2), pl.ds(base_w + ci * P, P)])

    def level_body(l, carry):
        parity = l & 1

        @pl.when(sid == 0)
        def _():
            pltpu.make_async_copy(
                tpk_hbm.at[pl.ds(l * TABLE_SIZE, TABLE_SIZE)],
                shared.at[pl.ds(parity * TABLE_SIZE, TABLE_SIZE)],
                sem_stage).wait()

        plsc.subcore_barrier()

        @pl.when(jnp.logical_and(sid == 0, l < NUM_LEVELS - 1))
        def _():
            nparity = parity ^ 1
            pltpu.async_copy(
                tpk_hbm.at[pl.ds((l + 1) * TABLE_SIZE, TABLE_SIZE)],
                shared.at[pl.ds(nparity * TABLE_SIZE, TABLE_SIZE)],
                sem_stage)

        hashp(l, parity, 0, idx0, wv0)
        pltpu.async_copy(shared.at[idx0], rows0, sem0)

        def pair(j, carry2):
            i0 = 2 * j
            hashp(l, parity, i0 + 1, idx1, wv1)
            pltpu.async_copy(shared.at[idx1], rows1, sem1)
            pltpu.make_async_copy(shared.at[idx0], rows0, sem0).wait()
            interp(l, i0, rows0, wv0)

            @pl.when(j < NCHUNK // 2 - 1)
            def _():
                hashp(l, parity, i0 + 2, idx0, wv0)
                pltpu.async_copy(shared.at[idx0], rows0, sem0)

            pltpu.make_async_copy(shared.at[idx1], rows1, sem1).wait()
            interp(l, i0 + 1, rows1, wv1)
            return carry2

        lax.fori_loop(0, NCHUNK // 2, pair, 0)
        return carry

    lax.fori_loop(0, NUM_LEVELS, level_body, 0)


@jax.jit
def kernel(x, hash_table):
    levels = jnp.arange(NUM_LEVELS)
    gf = jnp.exp((jnp.log(float(MAX_RESOLUTION)) - jnp.log(float(MIN_RESOLUTION)))
                 / (NUM_LEVELS - 1))
    scalings = jnp.floor(MIN_RESOLUTION * gf ** levels).astype(jnp.float32)
    scal_splat = jnp.broadcast_to(scalings[:, None], (NUM_LEVELS, 16)).reshape(-1)
    xt = x.T.reshape(-1)
    u = jax.lax.bitcast_convert_type(hash_table, jnp.uint32)
    r = (u + 0x7FFF + ((u >> 16) & 1)) >> 16
    tpk = jax.lax.bitcast_convert_type(r[:, 0] | (r[:, 1] << 16), jnp.int32)

    mesh = plsc.VectorSubcoreMesh(core_axis_name="c", subcore_axis_name="s")
    run = pl.kernel(
        _body,
        out_type=jax.ShapeDtypeStruct((OUTD, N_POINTS), jnp.float32),
        mesh=mesh,
        scratch_types=[
            pltpu.VMEM_SHARED((2 * TABLE_SIZE,), jnp.int32),
            pltpu.VMEM((3 * NPW,), jnp.float32),
            pltpu.VMEM((NUM_LEVELS * 16,), jnp.float32),
            pltpu.VMEM((3 * P,), jnp.float32),
            pltpu.VMEM((3 * P,), jnp.float32),
            pltpu.VMEM((NIDX,), jnp.int32),
            pltpu.VMEM((NIDX,), jnp.int32),
            pltpu.VMEM((NIDX,), jnp.int32),
            pltpu.VMEM((NIDX,), jnp.int32),
            pltpu.VMEM((2, P), jnp.float32),
            pltpu.SemaphoreType.DMA,
            pltpu.SemaphoreType.DMA,
            pltpu.SemaphoreType.DMA,
        ],
        compiler_params=pltpu.CompilerParams(needs_layout_passes=False),
    )
    out = run(xt, tpk, scal_splat)
    return out.T


# speedup vs baseline: 1.1237x; 1.1237x over previous
import jax
import jax.numpy as jnp
import numpy as np
from jax import lax
from jax.experimental import pallas as pl
from jax.experimental.pallas import tpu as pltpu
from jax.experimental.pallas import tpu_sc as plsc

TABLE_SIZE = 524288
NUM_LEVELS = 16
MIN_RESOLUTION = 16
MAX_RESOLUTION = 2048
FEATURE_DIM = 2
N_POINTS = 131072

_K1 = int(np.uint32(2654435761).view(np.int32))
_K2 = 805459861
_MASK = TABLE_SIZE - 1

NW = 32
NPW = N_POINTS // NW
P = 512
NCHUNK = NPW // P
NG = P // 16
OUTD = NUM_LEVELS * FEATURE_DIM
NIDX = 8 * P


def _body(xt_hbm, tpk_hbm, scal_hbm, out_hbm,
          shared, xv, sv, wv0, wv1, idx0, idx1, rows0, rows1, outv,
          sem0, sem1, sem_stage):
    cid = lax.axis_index("c")
    sid = lax.axis_index("s")
    wid = sid * 2 + cid
    base_w = wid * NPW

    pltpu.sync_copy(scal_hbm, sv)
    for c in range(3):
        pltpu.sync_copy(xt_hbm.at[pl.ds(c * N_POINTS + base_w, NPW)],
                        xv.at[pl.ds(c * NPW, NPW)])

    @pl.when(sid == 0)
    def _():
        pltpu.async_copy(tpk_hbm.at[pl.ds(0, TABLE_SIZE)],
                         shared.at[pl.ds(0, TABLE_SIZE)], sem_stage)

    def hashp(l, parity, ci, idxb, wvb):
        off = ci * P
        sbase = parity * TABLE_SIZE

        @plsc.parallel_loop(0, NG, 1, unroll=2)
        def grp1(g):
            p0 = off + g * 16
            x0 = xv[pl.ds(p0, 16)]
            x1 = xv[pl.ds(NPW + p0, 16)]
            x2 = xv[pl.ds(2 * NPW + p0, 16)]
            s = sv[pl.ds(l * 16, 16)]
            sx0 = x0 * s
            sx1 = x1 * s
            sx2 = x2 * s
            f0 = sx0.astype(jnp.int32)
            f1 = sx1.astype(jnp.int32)
            f2 = sx2.astype(jnp.int32)
            ff0 = f0.astype(jnp.float32)
            ff1 = f1.astype(jnp.float32)
            ff2 = f2.astype(jnp.float32)
            c0 = jnp.where(sx0 > ff0, f0 + 1, f0)
            c1 = jnp.where(sx1 > ff1, f1 + 1, f1)
            c2 = jnp.where(sx2 > ff2, f2 + 1, f2)
            wvb[pl.ds(0 * P + g * 16, 16)] = sx0 - ff0
            wvb[pl.ds(1 * P + g * 16, 16)] = sx1 - ff1
            wvb[pl.ds(2 * P + g * 16, 16)] = sx2 - ff2
            tyc = c1 * _K1
            tyf = f1 * _K1
            tzc = c2 * _K2
            tzf = f2 * _K2
            xy_cc = c0 ^ tyc
            xy_cf = c0 ^ tyf
            xy_fc = f0 ^ tyc
            xy_ff = f0 ^ tyf
            hs = [
                xy_cc ^ tzc, xy_cc ^ tzf, xy_cf ^ tzc, xy_fc ^ tzc,
                xy_cf ^ tzf, xy_fc ^ tzf, xy_ff ^ tzc, xy_ff ^ tzf,
            ]
            for k in range(8):
                idxb[pl.ds(k * P + g * 16, 16)] = (hs[k] & _MASK) + sbase

    def interp(l, ci, rowsb, wvb):
        @plsc.parallel_loop(0, NG, 1, unroll=2)
        def grp2(g):
            wx = wvb[pl.ds(0 * P + g * 16, 16)]
            wy = wvb[pl.ds(1 * P + g * 16, 16)]
            wz = wvb[pl.ds(2 * P + g * 16, 16)]
            fpk = [rowsb[pl.ds(k * P + g * 16, 16)] for k in range(8)]
            for ch in range(2):
                if ch == 0:
                    f = [plsc.bitcast(v << 16, jnp.float32) for v in fpk]
                else:
                    f = [plsc.bitcast(v & (-65536), jnp.float32) for v in fpk]
                f03 = f[3] + wx * (f[0] - f[3])
                f12 = f[2] + wx * (f[1] - f[2])
                f56 = f[6] + wx * (f[5] - f[6])
                f47 = f[7] + wx * (f[4] - f[7])
                f0312 = f12 + wy * (f03 - f12)
                f4756 = f56 + wy * (f47 - f56)
                enc = f4756 + wz * (f0312 - f4756)
                outv[ch, pl.ds(g * 16, 16)] = enc
        pltpu.sync_copy(
            outv, out_hbm.at[pl.ds(2 * l, 2), pl.ds(base_w + ci * P, P)])

    def level_body(l, carry):
        parity = l & 1

        @pl.when(sid == 0)
        def _():
            pltpu.make_async_copy(
                tpk_hbm.at[pl.ds(l * TABLE_SIZE, TABLE_SIZE)],
                shared.at[pl.ds(parity * TABLE_SIZE, TABLE_SIZE)],
                sem_stage).wait()

        plsc.subcore_barrier()

        @pl.when(jnp.logical_and(sid == 0, l < NUM_LEVELS - 1))
        def _():
            nparity = parity ^ 1
            pltpu.async_copy(
                tpk_hbm.at[pl.ds((l + 1) * TABLE_SIZE, TABLE_SIZE)],
                shared.at[pl.ds(nparity * TABLE_SIZE, TABLE_SIZE)],
                sem_stage)

        hashp(l, parity, 0, idx0, wv0)
        pltpu.async_copy(shared.at[idx0], rows0, sem0)

        def pair(j, carry2):
            i0 = 2 * j
            hashp(l, parity, i0 + 1, idx1, wv1)
            pltpu.async_copy(shared.at[idx1], rows1, sem1)
            pltpu.make_async_copy(shared.at[idx0], rows0, sem0).wait()
            interp(l, i0, rows0, wv0)

            @pl.when(j < NCHUNK // 2 - 1)
            def _():
                hashp(l, parity, i0 + 2, idx0, wv0)
                pltpu.async_copy(shared.at[idx0], rows0, sem0)

            pltpu.make_async_copy(shared.at[idx1], rows1, sem1).wait()
            interp(l, i0 + 1, rows1, wv1)
            return carry2

        lax.fori_loop(0, NCHUNK // 2, pair, 0)
        return carry

    lax.fori_loop(0, NUM_LEVELS, level_body, 0)


@jax.jit
def kernel(x, hash_table):
    levels = jnp.arange(NUM_LEVELS)
    gf = jnp.exp((jnp.log(float(MAX_RESOLUTION)) - jnp.log(float(MIN_RESOLUTION)))
                 / (NUM_LEVELS - 1))
    scalings = jnp.floor(MIN_RESOLUTION * gf ** levels).astype(jnp.float32)
    scal_splat = jnp.broadcast_to(scalings[:, None], (NUM_LEVELS, 16)).reshape(-1)
    xt = x.T.reshape(-1)
    u = jax.lax.bitcast_convert_type(hash_table, jnp.uint32)
    r = (u + 0x7FFF + ((u >> 16) & 1)) >> 16
    tpk = jax.lax.bitcast_convert_type(r[:, 0] | (r[:, 1] << 16), jnp.int32)

    mesh = plsc.VectorSubcoreMesh(core_axis_name="c", subcore_axis_name="s")
    run = pl.kernel(
        _body,
        out_type=jax.ShapeDtypeStruct((OUTD, N_POINTS), jnp.float32),
        mesh=mesh,
        scratch_types=[
            pltpu.VMEM_SHARED((2 * TABLE_SIZE,), jnp.int32),
            pltpu.VMEM((3 * NPW,), jnp.float32),
            pltpu.VMEM((NUM_LEVELS * 16,), jnp.float32),
            pltpu.VMEM((3 * P,), jnp.float32),
            pltpu.VMEM((3 * P,), jnp.float32),
            pltpu.VMEM((NIDX,), jnp.int32),
            pltpu.VMEM((NIDX,), jnp.int32),
            pltpu.VMEM((NIDX,), jnp.int32),
            pltpu.VMEM((NIDX,), jnp.int32),
            pltpu.VMEM((2, P), jnp.float32),
            pltpu.SemaphoreType.DMA,
            pltpu.SemaphoreType.DMA,
            pltpu.SemaphoreType.DMA,
        ],
        compiler_params=pltpu.CompilerParams(needs_layout_passes=False),
    )
    out = run(xt, tpk, scal_splat)
    return out.T
